# baseline (speedup 1.0000x reference)
import functools

import jax
import jax.numpy as jnp
from jax.experimental import pallas as pl
from jax.experimental.pallas import tpu as pltpu

N_EXP = 64
K = 2


def _process_block(logits, idx_ref, wts_ref):
    iota = jax.lax.broadcasted_iota(jnp.int32, logits.shape, 1)

    rowmax = jnp.max(logits, axis=1, keepdims=True)
    i1 = jnp.min(jnp.where(logits == rowmax, iota, N_EXP), axis=1)
    hit1 = iota == i1[:, None]
    masked = jnp.where(hit1, -jnp.inf, logits)
    m2 = jnp.max(masked, axis=1, keepdims=True)
    i2 = jnp.min(jnp.where(masked == m2, iota, N_EXP), axis=1)
    hit2 = iota == i2[:, None]

    ex = jnp.exp(logits - rowmax)
    denom = jnp.sum(ex, axis=1, keepdims=True)
    rdenom = 1.0 / denom
    probs = ex * rdenom
    p1 = rdenom
    p2 = jnp.exp(m2 - rowmax) * rdenom

    s = p1 + p2 + 1e-8
    idx_ref[...] = jnp.concatenate([i1[:, None], i2[:, None]], axis=1)
    wts_ref[...] = jnp.concatenate([p1 / s, p2 / s], axis=1)

    cnt = jnp.sum(hit1.astype(jnp.float32) + hit2.astype(jnp.float32),
                  axis=0)[None, :]
    psum = jnp.sum(probs, axis=0)[None, :]
    return cnt, psum


def _router_kernel(xa_ref, xb_ref, w_ref,
                   idxa_ref, wtsa_ref, idxb_ref, wtsb_ref,
                   cnt_ref, psum_ref, aux_ref, *, n_tokens, n_steps):
    step = pl.program_id(0)
    wt = w_ref[...].T

    logits_a = jnp.dot(xa_ref[...], wt, preferred_element_type=jnp.float32)
    cnt_a, psum_a = _process_block(logits_a, idxa_ref, wtsa_ref)

    logits_b = jnp.dot(xb_ref[...], wt, preferred_element_type=jnp.float32)
    cnt_b, psum_b = _process_block(logits_b, idxb_ref, wtsb_ref)

    cnt_blk = cnt_a + cnt_b
    psum_blk = psum_a + psum_b

    @pl.when(step == 0)
    def _init():
        cnt_ref[...] = cnt_blk
        psum_ref[...] = psum_blk

    @pl.when(step != 0)
    def _acc():
        cnt_ref[...] += cnt_blk
        psum_ref[...] += psum_blk

    @pl.when(step == n_steps - 1)
    def _finalize():
        f = cnt_ref[...] / (n_tokens * K)
        p = psum_ref[...] / n_tokens
        aux_ref[...] = (N_EXP * jnp.sum(f * p)).reshape(1, 1)


def kernel(x, W):
    b, s, d = x.shape
    n_tokens = b * s
    x_flat = x.reshape(n_tokens, d)

    block_t = 1024
    half_blocks = n_tokens // (2 * block_t)
    n_steps = half_blocks

    grid_spec = pl.GridSpec(
        grid=(n_steps,),
        in_specs=[
            pl.BlockSpec((block_t, d), lambda i: (i, 0)),
            pl.BlockSpec((block_t, d), lambda i, hb=half_blocks: (i + hb, 0)),
            pl.BlockSpec((N_EXP, d), lambda i: (0, 0)),
        ],
        out_specs=[
            pl.BlockSpec((block_t, K), lambda i: (i, 0)),
            pl.BlockSpec((block_t, K), lambda i: (i, 0)),
            pl.BlockSpec((block_t, K), lambda i: (i, 0)),
            pl.BlockSpec((block_t, K), lambda i: (i, 0)),
            pl.BlockSpec((1, N_EXP), lambda i: (0, 0)),
            pl.BlockSpec((1, N_EXP), lambda i: (0, 0)),
            pl.BlockSpec((1, 1), lambda i: (0, 0)),
        ],
    )

    idxa, wtsa, idxb, wtsb, _cnt, _psum, aux = pl.pallas_call(
        functools.partial(_router_kernel, n_tokens=n_tokens, n_steps=n_steps),
        grid_spec=grid_spec,
        out_shape=[
            jax.ShapeDtypeStruct((n_tokens // 2, K), jnp.int32),
            jax.ShapeDtypeStruct((n_tokens // 2, K), jnp.float32),
            jax.ShapeDtypeStruct((n_tokens // 2, K), jnp.int32),
            jax.ShapeDtypeStruct((n_tokens // 2, K), jnp.float32),
            jax.ShapeDtypeStruct((1, N_EXP), jnp.float32),
            jax.ShapeDtypeStruct((1, N_EXP), jnp.float32),
            jax.ShapeDtypeStruct((1, 1), jnp.float32),
        ],
        compiler_params=pltpu.CompilerParams(
            dimension_semantics=("arbitrary",),
        ),
    )(x_flat, x_flat, W)

    idx = jnp.concatenate([idxa, idxb], axis=0)
    wts = jnp.concatenate([wtsa, wtsb], axis=0)
    return (idx, wts, aux[0, 0])


# speedup vs baseline: 1.0021x; 1.0021x over previous
import functools

import jax
import jax.numpy as jnp
from jax.experimental import pallas as pl
from jax.experimental.pallas import tpu as pltpu

N_EXP = 64
K = 2


def _router_kernel(x_ref, w_ref, idx_ref, wts_ref, cnt_ref, psum_ref, aux_ref,
                   *, n_tokens, n_steps):
    step = pl.program_id(0)

    logits = jax.lax.dot_general(
        x_ref[...], w_ref[...],
        (((1,), (1,)), ((), ())),
        preferred_element_type=jnp.float32)

    iota = jax.lax.broadcasted_iota(jnp.int32, logits.shape, 1)

    rowmax = jnp.max(logits, axis=1, keepdims=True)
    i1 = jnp.min(jnp.where(logits == rowmax, iota, N_EXP), axis=1)
    hit1 = iota == i1[:, None]
    masked = jnp.where(hit1, -jnp.inf, logits)
    m2 = jnp.max(masked, axis=1, keepdims=True)
    i2 = jnp.min(jnp.where(masked == m2, iota, N_EXP), axis=1)
    hit2 = iota == i2[:, None]

    ex = jnp.exp(logits - rowmax)
    denom = jnp.sum(ex, axis=1, keepdims=True)
    rdenom = 1.0 / denom
    probs = ex * rdenom
    p1 = rdenom
    p2 = jnp.exp(m2 - rowmax) * rdenom

    s = p1 + p2 + 1e-8
    idx_ref[...] = jnp.concatenate([i1[:, None], i2[:, None]], axis=1)
    wts_ref[...] = jnp.concatenate([p1 / s, p2 / s], axis=1)

    cnt_blk = jnp.sum(hit1.astype(jnp.float32) + hit2.astype(jnp.float32),
                      axis=0)[None, :]
    psum_blk = jnp.sum(probs, axis=0)[None, :]

    @pl.when(step == 0)
    def _init():
        cnt_ref[...] = cnt_blk
        psum_ref[...] = psum_blk

    @pl.when(step != 0)
    def _acc():
        cnt_ref[...] += cnt_blk
        psum_ref[...] += psum_blk

    @pl.when(step == n_steps - 1)
    def _finalize():
        f = cnt_ref[...] / (n_tokens * K)
        p = psum_ref[...] / n_tokens
        aux_ref[...] = (N_EXP * jnp.sum(f * p)).reshape(1, 1)


def kernel(x, W):
    b, s, d = x.shape
    n_tokens = b * s
    x_flat = x.reshape(n_tokens, d)

    block_t = 2048
    n_steps = n_tokens // block_t

    grid_spec = pl.GridSpec(
        grid=(n_steps,),
        in_specs=[
            pl.BlockSpec((block_t, d), lambda i: (i, 0)),
            pl.BlockSpec((N_EXP, d), lambda i: (0, 0)),
        ],
        out_specs=[
            pl.BlockSpec((block_t, K), lambda i: (i, 0)),
            pl.BlockSpec((block_t, K), lambda i: (i, 0)),
            pl.BlockSpec((1, N_EXP), lambda i: (0, 0)),
            pl.BlockSpec((1, N_EXP), lambda i: (0, 0)),
            pl.BlockSpec((1, 1), lambda i: (0, 0)),
        ],
    )

    idx, wts, _cnt, _psum, aux = pl.pallas_call(
        functools.partial(_router_kernel, n_tokens=n_tokens, n_steps=n_steps),
        grid_spec=grid_spec,
        out_shape=[
            jax.ShapeDtypeStruct((n_tokens, K), jnp.int32),
            jax.ShapeDtypeStruct((n_tokens, K), jnp.float32),
            jax.ShapeDtypeStruct((1, N_EXP), jnp.float32),
            jax.ShapeDtypeStruct((1, N_EXP), jnp.float32),
            jax.ShapeDtypeStruct((1, 1), jnp.float32),
        ],
        compiler_params=pltpu.CompilerParams(
            dimension_semantics=("arbitrary",),
        ),
    )(x_flat, W)

    return (idx, wts, aux[0, 0])
